# initial kernel scaffold (unmeasured)
import jax
import jax.numpy as jnp
from jax import lax
from jax.experimental import pallas as pl
from jax.experimental.pallas import tpu as pltpu

N_DEV = 32
N_SRC = 16
B, SQ, DM = 2, 128, 512
HL, DH = 4, 64
KV_LOC = 128
QB = 64


def kernel(x, Wq, K_ext, V_ext, Wo):
    def body(x_ref, wq_ref, ko_ref, vo_ref, wo_ref, out_ref,
             k_recv, v_recv, p_recv, p_own,
             k_send_sems, v_send_sems, p_send_sems,
             k_recv_sems, v_recv_sems, p_recv_sems):
        my = lax.axis_index("i")
        my_slot = my // 2
        even = lax.rem(my, 2) == 0

        def kv_desc(src_ref, recv_buf, send_sems, recv_sems, head_start,
                    dst, slot):
            return pltpu.make_async_remote_copy(
                src_ref=src_ref.at[:, :, pl.ds(head_start, HL), :],
                dst_ref=recv_buf.at[slot],
                send_sem=send_sems.at[dst],
                recv_sem=recv_sems.at[slot],
                device_id=(dst,),
                device_id_type=pl.DeviceIdType.MESH,
            )

        @pl.when(even)
        def _():
            for off in range(1, N_DEV):
                dst = lax.rem(my + off, N_DEV)
                kv_desc(ko_ref, k_recv, k_send_sems, k_recv_sems,
                        dst * HL, dst, my_slot).start()
                kv_desc(vo_ref, v_recv, v_send_sems, v_recv_sems,
                        dst * HL, dst, my_slot).start()
            pltpu.make_async_copy(
                ko_ref.at[:, :, pl.ds(my * HL, HL), :],
                k_recv.at[my_slot], k_recv_sems.at[my_slot]).start()
            pltpu.make_async_copy(
                vo_ref.at[:, :, pl.ds(my * HL, HL), :],
                v_recv.at[my_slot], v_recv_sems.at[my_slot]).start()

        q = jnp.dot(x_ref[...].reshape(B * SQ, DM), wq_ref[...],
                    preferred_element_type=jnp.float32)
        q4 = q.reshape(B, SQ, HL, DH)

        for m in range(N_SRC):
            kv_desc(ko_ref, k_recv, k_send_sems, k_recv_sems,
                    0, 0, m).wait_recv()
            kv_desc(vo_ref, v_recv, v_send_sems, v_recv_sems,
                    0, 0, m).wait_recv()

        ctx_blocks = []
        for qb in range(2):
            sl = slice(qb * QB, (qb + 1) * QB)
            qb_t = q4[:, sl].transpose(0, 2, 1, 3).reshape(B * HL, QB, DH)
            kb = k_recv[:, :, sl]
            kb = kb.transpose(1, 3, 0, 2, 4).reshape(B * HL, N_SRC * QB, DH)
            vb = v_recv[:, :, sl]
            vb = vb.transpose(1, 3, 0, 2, 4).reshape(B * HL, N_SRC * QB, DH)
            s = lax.dot_general(
                qb_t, kb, (((2,), (2,)), ((0,), (0,))),
                preferred_element_type=jnp.float32) * 0.125
            mx = jnp.max(s, axis=-1, keepdims=True)
            w = jnp.exp(s - mx)
            w = w / jnp.sum(w, axis=-1, keepdims=True)
            ctx = lax.dot_general(
                w, vb, (((2,), (1,)), ((0,), (0,))),
                preferred_element_type=jnp.float32)
            ctx_blocks.append(ctx.reshape(B, HL, QB, DH).transpose(0, 2, 1, 3))
        ctxf = jnp.concatenate(ctx_blocks, axis=1)
        partial = jnp.dot(ctxf.reshape(B * SQ, HL * DH), wo_ref[...],
                          preferred_element_type=jnp.float32)
        p_own[...] = partial.reshape(B, SQ, DM)

        def p_desc(off, slot):
            return pltpu.make_async_remote_copy(
                src_ref=p_own,
                dst_ref=p_recv.at[slot],
                send_sem=p_send_sems.at[off - 1],
                recv_sem=p_recv_sems.at[slot],
                device_id=(lax.rem(my + off, N_DEV),),
                device_id_type=pl.DeviceIdType.MESH,
            )

        for off in range(1, N_DEV):
            p_desc(off, N_DEV - 1 - off).start()

        total = p_own[...]
        for s_ in range(N_DEV - 1):
            p_desc(1, s_).wait_recv()
            total = total + p_recv[s_]
        out_ref[...] = total

        @pl.when(even)
        def _():
            for off in range(1, N_DEV):
                dst = lax.rem(my + off, N_DEV)
                kv_desc(ko_ref, k_recv, k_send_sems, k_recv_sems,
                        0, dst, my_slot).wait_send()
                kv_desc(vo_ref, v_recv, v_send_sems, v_recv_sems,
                        0, dst, my_slot).wait_send()
        for off in range(1, N_DEV):
            p_desc(off, 0).wait_send()

    return pl.pallas_call(
        body,
        out_shape=jax.ShapeDtypeStruct((B, SQ, DM), jnp.float32),
        in_specs=[
            pl.BlockSpec(memory_space=pltpu.VMEM),
            pl.BlockSpec(memory_space=pltpu.VMEM),
            pl.BlockSpec(memory_space=pltpu.ANY),
            pl.BlockSpec(memory_space=pltpu.ANY),
            pl.BlockSpec(memory_space=pltpu.VMEM),
        ],
        out_specs=pl.BlockSpec(memory_space=pltpu.VMEM),
        scratch_shapes=[
            pltpu.VMEM((N_SRC, B, KV_LOC, HL, DH), jnp.float32),
            pltpu.VMEM((N_SRC, B, KV_LOC, HL, DH), jnp.float32),
            pltpu.VMEM((N_DEV - 1, B, SQ, DM), jnp.float32),
            pltpu.VMEM((B, SQ, DM), jnp.float32),
            pltpu.SemaphoreType.DMA((N_DEV,)),
            pltpu.SemaphoreType.DMA((N_DEV,)),
            pltpu.SemaphoreType.DMA((N_DEV - 1,)),
            pltpu.SemaphoreType.DMA((N_SRC,)),
            pltpu.SemaphoreType.DMA((N_SRC,)),
            pltpu.SemaphoreType.DMA((N_DEV - 1,)),
        ],
        compiler_params=pltpu.CompilerParams(
            collective_id=0,
            vmem_limit_bytes=112 * 1024 * 1024,
        ),
    )(x, Wq, K_ext, V_ext, Wo)


# baseline (device time: 624312 ns/iter reference)
import jax
import jax.numpy as jnp
from jax import lax
from jax.experimental import pallas as pl
from jax.experimental.pallas import tpu as pltpu

N_DEV = 32
N_SRC = 16
B, SQ, DM = 2, 128, 512
HL, DH = 4, 64
KV_LOC = 128
QB = 64


def kernel(x, Wq, K_ext, V_ext, Wo):
    def body(x_ref, wq_ref, ko_ref, vo_ref, wo_ref, out_ref,
             k_recv, v_recv, p_recv, p_own,
             k_send_sems, v_send_sems, p_send_sems,
             k_recv_sems, v_recv_sems, p_recv_sems):
        my = lax.axis_index("i")
        my_slot = my // 2
        even = lax.rem(my, 2) == 0

        def kv_desc(src_ref, recv_buf, send_sems, recv_sems, head_start,
                    dst, slot):
            return pltpu.make_async_remote_copy(
                src_ref=src_ref.at[:, :, pl.ds(head_start, HL), :],
                dst_ref=recv_buf.at[slot],
                send_sem=send_sems.at[dst],
                recv_sem=recv_sems.at[slot],
                device_id=(dst,),
                device_id_type=pl.DeviceIdType.MESH,
            )

        @pl.when(even)
        def _():
            for off in range(1, N_DEV):
                dst = lax.rem(my + off, N_DEV)
                kv_desc(ko_ref, k_recv, k_send_sems, k_recv_sems,
                        dst * HL, dst, my_slot).start()
                kv_desc(vo_ref, v_recv, v_send_sems, v_recv_sems,
                        dst * HL, dst, my_slot).start()
            pltpu.make_async_copy(
                ko_ref.at[:, :, pl.ds(my * HL, HL), :],
                k_recv.at[my_slot], k_recv_sems.at[my_slot]).start()
            pltpu.make_async_copy(
                vo_ref.at[:, :, pl.ds(my * HL, HL), :],
                v_recv.at[my_slot], v_recv_sems.at[my_slot]).start()

        q = jnp.dot(x_ref[...].reshape(B * SQ, DM), wq_ref[...],
                    preferred_element_type=jnp.float32)
        q4 = q.reshape(B, SQ, HL, DH)

        for m in range(N_SRC):
            kv_desc(ko_ref, k_recv, k_send_sems, k_recv_sems,
                    0, 0, m).wait_recv()
            kv_desc(vo_ref, v_recv, v_send_sems, v_recv_sems,
                    0, 0, m).wait_recv()

        ctx_blocks = []
        for qb in range(2):
            sl = slice(qb * QB, (qb + 1) * QB)
            qb_t = q4[:, sl].transpose(0, 2, 1, 3).reshape(B * HL, QB, DH)
            kb = k_recv[:, :, sl]
            kb = kb.transpose(1, 3, 0, 2, 4).reshape(B * HL, N_SRC * QB, DH)
            vb = v_recv[:, :, sl]
            vb = vb.transpose(1, 3, 0, 2, 4).reshape(B * HL, N_SRC * QB, DH)
            s = lax.dot_general(
                qb_t, kb, (((2,), (2,)), ((0,), (0,))),
                preferred_element_type=jnp.float32) * 0.125
            mx = jnp.max(s, axis=-1, keepdims=True)
            w = jnp.exp(s - mx)
            w = w / jnp.sum(w, axis=-1, keepdims=True)
            ctx = lax.dot_general(
                w, vb, (((2,), (1,)), ((0,), (0,))),
                preferred_element_type=jnp.float32)
            ctx_blocks.append(ctx.reshape(B, HL, QB, DH).transpose(0, 2, 1, 3))
        ctxf = jnp.concatenate(ctx_blocks, axis=1)
        partial = jnp.dot(ctxf.reshape(B * SQ, HL * DH), wo_ref[...],
                          preferred_element_type=jnp.float32)
        p_own[...] = partial.reshape(B, SQ, DM)

        def p_desc(off, slot):
            return pltpu.make_async_remote_copy(
                src_ref=p_own,
                dst_ref=p_recv.at[slot],
                send_sem=p_send_sems.at[off - 1],
                recv_sem=p_recv_sems.at[slot],
                device_id=(lax.rem(my + off, N_DEV),),
                device_id_type=pl.DeviceIdType.MESH,
            )

        for off in range(1, N_DEV):
            p_desc(off, N_DEV - 1 - off).start()

        total = p_own[...]
        for s_ in range(N_DEV - 1):
            p_desc(1, s_).wait_recv()
            total = total + p_recv[s_]
        out_ref[...] = total

        @pl.when(even)
        def _():
            for off in range(1, N_DEV):
                dst = lax.rem(my + off, N_DEV)
                kv_desc(ko_ref, k_recv, k_send_sems, k_recv_sems,
                        0, dst, my_slot).wait_send()
                kv_desc(vo_ref, v_recv, v_send_sems, v_recv_sems,
                        0, dst, my_slot).wait_send()
        for off in range(1, N_DEV):
            p_desc(off, 0).wait_send()

    return pl.pallas_call(
        body,
        out_shape=jax.ShapeDtypeStruct((B, SQ, DM), jnp.float32),
        in_specs=[
            pl.BlockSpec(memory_space=pltpu.VMEM),
            pl.BlockSpec(memory_space=pltpu.VMEM),
            pl.BlockSpec(memory_space=pltpu.HBM),
            pl.BlockSpec(memory_space=pltpu.HBM),
            pl.BlockSpec(memory_space=pltpu.VMEM),
        ],
        out_specs=pl.BlockSpec(memory_space=pltpu.VMEM),
        scratch_shapes=[
            pltpu.VMEM((N_SRC, B, KV_LOC, HL, DH), jnp.float32),
            pltpu.VMEM((N_SRC, B, KV_LOC, HL, DH), jnp.float32),
            pltpu.VMEM((N_DEV - 1, B, SQ, DM), jnp.float32),
            pltpu.VMEM((B, SQ, DM), jnp.float32),
            pltpu.SemaphoreType.DMA((N_DEV,)),
            pltpu.SemaphoreType.DMA((N_DEV,)),
            pltpu.SemaphoreType.DMA((N_DEV - 1,)),
            pltpu.SemaphoreType.DMA((N_SRC,)),
            pltpu.SemaphoreType.DMA((N_SRC,)),
            pltpu.SemaphoreType.DMA((N_DEV - 1,)),
        ],
        compiler_params=pltpu.CompilerParams(
            vmem_limit_bytes=112 * 1024 * 1024,
        ),
    )(x, Wq, K_ext, V_ext, Wo)


# device time: 446436 ns/iter; 1.3984x vs baseline; 1.3984x over previous
import jax
import jax.numpy as jnp
from jax import lax
from jax.experimental import pallas as pl
from jax.experimental.pallas import tpu as pltpu

N_DEV = 32
N_SRC = 16
B, SQ, DM = 2, 128, 512
HL, DH = 4, 64
KV_LOC = 128
QB = 64


def kernel(x, Wq, K_ext, V_ext, Wo):
    def body(x_ref, wq_ref, ko_ref, vo_ref, wo_ref, out_ref,
             k_recv, v_recv, p_own, rs_recv,
             k_send_sems, v_send_sems,
             k_recv_sems, v_recv_sems,
             rs_send_sems, rs_recv_sems, ag_send_sems, ag_recv_sems):
        my = lax.axis_index("i")
        my_slot = my // 2
        even = lax.rem(my, 2) == 0

        def kv_desc(src_ref, recv_buf, send_sems, recv_sems, head_start,
                    dst, slot):
            return pltpu.make_async_remote_copy(
                src_ref=src_ref.at[:, :, pl.ds(head_start, HL), :],
                dst_ref=recv_buf.at[slot],
                send_sem=send_sems.at[dst],
                recv_sem=recv_sems.at[slot],
                device_id=(dst,),
                device_id_type=pl.DeviceIdType.MESH,
            )

        @pl.when(even)
        def _():
            for off in range(1, N_DEV):
                dst = lax.rem(my + off, N_DEV)
                kv_desc(ko_ref, k_recv, k_send_sems, k_recv_sems,
                        dst * HL, dst, my_slot).start()
                kv_desc(vo_ref, v_recv, v_send_sems, v_recv_sems,
                        dst * HL, dst, my_slot).start()
            pltpu.make_async_copy(
                ko_ref.at[:, :, pl.ds(my * HL, HL), :],
                k_recv.at[my_slot], k_recv_sems.at[my_slot]).start()
            pltpu.make_async_copy(
                vo_ref.at[:, :, pl.ds(my * HL, HL), :],
                v_recv.at[my_slot], v_recv_sems.at[my_slot]).start()

        q = jnp.dot(x_ref[...].reshape(B * SQ, DM), wq_ref[...],
                    preferred_element_type=jnp.float32)
        q4 = q.reshape(B, SQ, HL, DH)

        for m in range(N_SRC):
            kv_desc(ko_ref, k_recv, k_send_sems, k_recv_sems,
                    0, 0, m).wait_recv()
            kv_desc(vo_ref, v_recv, v_send_sems, v_recv_sems,
                    0, 0, m).wait_recv()

        ctx_blocks = []
        for qb in range(2):
            sl = slice(qb * QB, (qb + 1) * QB)
            qb_t = q4[:, sl].transpose(0, 2, 1, 3).reshape(B * HL, QB, DH)
            kb = k_recv[:, :, sl]
            kb = kb.transpose(1, 3, 0, 2, 4).reshape(B * HL, N_SRC * QB, DH)
            vb = v_recv[:, :, sl]
            vb = vb.transpose(1, 3, 0, 2, 4).reshape(B * HL, N_SRC * QB, DH)
            s = lax.dot_general(
                qb_t, kb, (((2,), (2,)), ((0,), (0,))),
                preferred_element_type=jnp.float32) * 0.125
            mx = jnp.max(s, axis=-1, keepdims=True)
            w = jnp.exp(s - mx)
            w = w / jnp.sum(w, axis=-1, keepdims=True)
            ctx = lax.dot_general(
                w, vb, (((2,), (1,)), ((0,), (0,))),
                preferred_element_type=jnp.float32)
            ctx_blocks.append(ctx.reshape(B, HL, QB, DH).transpose(0, 2, 1, 3))
        ctxf = jnp.concatenate(ctx_blocks, axis=1)
        partial = jnp.dot(ctxf.reshape(B * SQ, HL * DH), wo_ref[...],
                          preferred_element_type=jnp.float32)
        p_own[...] = partial

        ROWS = B * SQ
        RS_OFF = [0, 128, 192, 224, 240]
        lo = my * 0
        seg = ROWS
        for s in range(5):
            half = seg // 2
            bit = jnp.bitwise_and(jnp.right_shift(my, s), 1)
            partner = jnp.bitwise_xor(my, 1 << s)
            keep_lo = lo + bit * half
            send_lo = lo + (1 - bit) * half
            rdma = pltpu.make_async_remote_copy(
                src_ref=p_own.at[pl.ds(send_lo, half)],
                dst_ref=rs_recv.at[pl.ds(RS_OFF[s], half)],
                send_sem=rs_send_sems.at[s],
                recv_sem=rs_recv_sems.at[s],
                device_id=(partner,),
                device_id_type=pl.DeviceIdType.MESH,
            )
            rdma.start()
            rdma.wait()
            p_own[pl.ds(keep_lo, half)] = (
                p_own[pl.ds(keep_lo, half)]
                + rs_recv[pl.ds(RS_OFF[s], half)])
            lo = keep_lo
            seg = half

        for s in reversed(range(5)):
            partner = jnp.bitwise_xor(my, 1 << s)
            rdma = pltpu.make_async_remote_copy(
                src_ref=p_own.at[pl.ds(lo, seg)],
                dst_ref=p_own.at[pl.ds(lo, seg)],
                send_sem=ag_send_sems.at[s],
                recv_sem=ag_recv_sems.at[s],
                device_id=(partner,),
                device_id_type=pl.DeviceIdType.MESH,
            )
            rdma.start()
            rdma.wait()
            bit = jnp.bitwise_and(jnp.right_shift(my, s), 1)
            lo = lo - bit * seg
            seg = seg * 2

        out_ref[...] = p_own[...].reshape(B, SQ, DM)

        @pl.when(even)
        def _():
            for off in range(1, N_DEV):
                dst = lax.rem(my + off, N_DEV)
                kv_desc(ko_ref, k_recv, k_send_sems, k_recv_sems,
                        0, dst, my_slot).wait_send()
                kv_desc(vo_ref, v_recv, v_send_sems, v_recv_sems,
                        0, dst, my_slot).wait_send()

    return pl.pallas_call(
        body,
        out_shape=jax.ShapeDtypeStruct((B, SQ, DM), jnp.float32),
        in_specs=[
            pl.BlockSpec(memory_space=pltpu.VMEM),
            pl.BlockSpec(memory_space=pltpu.VMEM),
            pl.BlockSpec(memory_space=pltpu.VMEM),
            pl.BlockSpec(memory_space=pltpu.VMEM),
            pl.BlockSpec(memory_space=pltpu.VMEM),
        ],
        out_specs=pl.BlockSpec(memory_space=pltpu.VMEM),
        scratch_shapes=[
            pltpu.VMEM((N_SRC, B, KV_LOC, HL, DH), jnp.float32),
            pltpu.VMEM((N_SRC, B, KV_LOC, HL, DH), jnp.float32),
            pltpu.VMEM((B * SQ, DM), jnp.float32),
            pltpu.VMEM((248, DM), jnp.float32),
            pltpu.SemaphoreType.DMA((N_DEV,)),
            pltpu.SemaphoreType.DMA((N_DEV,)),
            pltpu.SemaphoreType.DMA((N_SRC,)),
            pltpu.SemaphoreType.DMA((N_SRC,)),
            pltpu.SemaphoreType.DMA((5,)),
            pltpu.SemaphoreType.DMA((5,)),
            pltpu.SemaphoreType.DMA((5,)),
            pltpu.SemaphoreType.DMA((5,)),
        ],
        compiler_params=pltpu.CompilerParams(
            vmem_limit_bytes=112 * 1024 * 1024,
        ),
    )(x, Wq, K_ext, V_ext, Wo)
